# baseline (speedup 1.0000x reference)
import functools

import numpy as np
import jax
import jax.numpy as jnp
from jax import lax
from jax.experimental import pallas as pl
from jax.experimental.pallas import tpu as pltpu
from jax.experimental.pallas import tpu_sc as plsc

N_RAYS = 4096
N_SAMPLES = 262144
HIDDEN = 128
L_FREQ = 6
D_IN = 3 + 3 * 2 * L_FREQ

LANES = 16
NC = 2
NS = 16
NW = NC * NS
CHUNK = N_SAMPLES // NW
GROUPS = CHUNK // LANES
RAYS_PER_W = N_RAYS // NW

D_PE = 56
BLK = 65536

_SIN_C3 = -0.16665682
_SIN_C5 = 0.0083123785
_SIN_C7 = -1.8492564e-4
_PI_F32 = 3.14159265358979
_INV_PI = 0.3183098861837907


def _fast_sin(a):
    n = jnp.floor(a * _INV_PI + 0.5)
    r = a - n * _PI_F32
    r2 = r * r
    p = _SIN_C7
    p = p * r2 + _SIN_C5
    p = p * r2 + _SIN_C3
    s = (p * r2) * r + r
    sbits = lax.shift_left(n.astype(jnp.int32), 31)
    return lax.bitcast_convert_type(
        lax.bitcast_convert_type(s, jnp.int32) ^ sbits, jnp.float32)


def _posenc_consts():
    Sts = np.zeros((16, 8), np.float32)
    for c in range(3):
        Sts[c, c] = 1.0
        Sts[8 + c, c] = 1.0
        Sts[12 + c, c] = 1.0
        Sts[12 + c, 4] = float(np.pi / 2)
    Sts[3, 4] = 1.0
    return Sts


_STS_CONST = _posenc_consts()


def _tc_mlp_body(pos_ref, st_ref, w1_ref, w2_ref, b2_ref,
                 w3_ref, b3_ref, out_ref):
    P = pos_ref[...]
    M = jnp.dot(st_ref[...], P, preferred_element_type=jnp.float32,
                precision=lax.Precision.HIGHEST)
    row8 = lax.broadcasted_iota(jnp.int32, (8, BLK), 0)
    groups = [M[0:8], _fast_sin(M[8:16])]
    for _ in range(L_FREQ - 1):
        G = groups[-1]
        Grot = jnp.roll(G, 4, axis=0)
        SS = G * Grot
        C = 1.0 - (G + G) * G
        X = jnp.where(row8 < 4, C, SS + SS)
        groups.append(jnp.roll(X, 4, axis=0))
    E = jnp.concatenate(groups, axis=0)
    h = jnp.maximum(
        jnp.dot(w1_ref[...], E.astype(jnp.bfloat16),
                preferred_element_type=jnp.float32).astype(jnp.bfloat16),
        jnp.bfloat16(0.0))
    h = jnp.maximum(
        jnp.dot(w2_ref[...], h,
                preferred_element_type=jnp.float32).astype(jnp.bfloat16)
        + b2_ref[...], jnp.bfloat16(0.0))
    z8 = jnp.dot(w3_ref[...], h, preferred_element_type=jnp.float32)
    z = z8[0:1, :] + b3_ref[...]
    sp = jnp.maximum(z, 0.0) + jnp.log(1.0 + jnp.exp(-jnp.abs(z)))
    out_ref[...] = sp * P[3:4, :]


def _tc_mlp(posP, Sts, W1t, W2t, b2c, W3r, b3c):
    grid = (N_SAMPLES // BLK,)
    return pl.pallas_call(
        _tc_mlp_body,
        grid=grid,
        in_specs=[
            pl.BlockSpec((8, BLK), lambda i: (0, i)),
            pl.BlockSpec((16, 8), lambda i: (0, 0)),
            pl.BlockSpec((HIDDEN, D_PE), lambda i: (0, 0)),
            pl.BlockSpec((HIDDEN, HIDDEN), lambda i: (0, 0)),
            pl.BlockSpec((HIDDEN, 1), lambda i: (0, 0)),
            pl.BlockSpec((8, HIDDEN), lambda i: (0, 0)),
            pl.BlockSpec((1, 1), lambda i: (0, 0)),
        ],
        out_specs=pl.BlockSpec((1, BLK), lambda i: (0, i)),
        out_shape=jax.ShapeDtypeStruct((1, N_SAMPLES), jnp.float32),
    )(posP, Sts, W1t, W2t, b2c, W3r, b3c)


@functools.cache
def _sc_kernels():
    mesh = plsc.VectorSubcoreMesh(core_axis_name="c", subcore_axis_name="s")

    @functools.partial(
        pl.kernel,
        out_type=jax.ShapeDtypeStruct((8, N_SAMPLES), jnp.float32),
        mesh=mesh,
        compiler_params=pltpu.CompilerParams(needs_layout_passes=False),
        scratch_types=[
            pltpu.VMEM((N_RAYS,), jnp.float32),
            pltpu.VMEM((N_RAYS,), jnp.float32),
            pltpu.VMEM((N_RAYS,), jnp.float32),
            pltpu.VMEM((N_RAYS,), jnp.float32),
            pltpu.VMEM((N_RAYS,), jnp.float32),
            pltpu.VMEM((N_RAYS,), jnp.float32),
            pltpu.VMEM((CHUNK,), jnp.int32),
            pltpu.VMEM((CHUNK,), jnp.float32),
            pltpu.VMEM((CHUNK,), jnp.float32),
            pltpu.VMEM((CHUNK,), jnp.float32),
            pltpu.VMEM((CHUNK,), jnp.float32),
            pltpu.VMEM((CHUNK,), jnp.float32),
            pltpu.VMEM((CHUNK,), jnp.float32),
            pltpu.VMEM((CHUNK,), jnp.float32),
            pltpu.SemaphoreType.DMA,
        ],
    )
    def sc_gather(ox_h, oy_h, oz_h, dx_h, dy_h, dz_h, ri_h, ts_h, te_h,
                  pos_h, ox_v, oy_v, oz_v, dx_v, dy_v, dz_v, ri_v, ts_v,
                  te_v, px_v, py_v, pz_v, dl_v, on_v, sem):
        wid = lax.axis_index("s") * NC + lax.axis_index("c")
        base = wid * CHUNK
        cps = [
            pltpu.async_copy(ox_h, ox_v, sem),
            pltpu.async_copy(oy_h, oy_v, sem),
            pltpu.async_copy(oz_h, oz_v, sem),
            pltpu.async_copy(dx_h, dx_v, sem),
            pltpu.async_copy(dy_h, dy_v, sem),
            pltpu.async_copy(dz_h, dz_v, sem),
            pltpu.async_copy(ri_h.at[pl.ds(base, CHUNK)], ri_v, sem),
            pltpu.async_copy(ts_h.at[pl.ds(base, CHUNK)], ts_v, sem),
            pltpu.async_copy(te_h.at[pl.ds(base, CHUNK)], te_v, sem),
        ]
        one = jnp.full((LANES,), 1.0, jnp.float32)

        def obody(i, c):
            o = i * (LANES * 8)
            for u in range(8):
                on_v[pl.ds(o + u * LANES, LANES)] = one
            return c

        lax.fori_loop(0, CHUNK // (LANES * 8), obody, 0)
        for cp in cps:
            cp.wait()

        def body(g):
            o = g * LANES
            idx = ri_v[pl.ds(o, LANES)]
            tsv = ts_v[pl.ds(o, LANES)]
            tev = te_v[pl.ds(o, LANES)]
            tm = (tsv + tev) * 0.5
            px_v[pl.ds(o, LANES)] = jnp.clip(
                plsc.load_gather(ox_v, [idx])
                + plsc.load_gather(dx_v, [idx]) * tm, -1.0, 1.0)
            py_v[pl.ds(o, LANES)] = jnp.clip(
                plsc.load_gather(oy_v, [idx])
                + plsc.load_gather(dy_v, [idx]) * tm, -1.0, 1.0)
            pz_v[pl.ds(o, LANES)] = jnp.clip(
                plsc.load_gather(oz_v, [idx])
                + plsc.load_gather(dz_v, [idx]) * tm, -1.0, 1.0)
            dl_v[pl.ds(o, LANES)] = tev - tsv

        plsc.parallel_loop(0, GROUPS, 1, unroll=4)(body)
        ops = [
            pltpu.async_copy(px_v, pos_h.at[0, pl.ds(base, CHUNK)], sem),
            pltpu.async_copy(py_v, pos_h.at[1, pl.ds(base, CHUNK)], sem),
            pltpu.async_copy(pz_v, pos_h.at[2, pl.ds(base, CHUNK)], sem),
            pltpu.async_copy(dl_v, pos_h.at[3, pl.ds(base, CHUNK)], sem),
            pltpu.async_copy(on_v, pos_h.at[4, pl.ds(base, CHUNK)], sem),
            pltpu.async_copy(on_v, pos_h.at[5, pl.ds(base, CHUNK)], sem),
            pltpu.async_copy(on_v, pos_h.at[6, pl.ds(base, CHUNK)], sem),
            pltpu.async_copy(on_v, pos_h.at[7, pl.ds(base, CHUNK)], sem),
        ]
        for cp in ops:
            cp.wait()

    @functools.partial(
        pl.kernel,
        out_type=jax.ShapeDtypeStruct((NW, N_RAYS), jnp.float32),
        mesh=mesh,
        compiler_params=pltpu.CompilerParams(needs_layout_passes=False),
        scratch_types=[
            pltpu.VMEM((1, CHUNK), jnp.float32),
            pltpu.VMEM((CHUNK,), jnp.int32),
            pltpu.VMEM((N_RAYS,), jnp.float32),
            pltpu.SemaphoreType.DMA,
        ],
    )
    def sc_segsum(sd_h, ri_h, part_h, sd_v, ri_v, acc_v, sem):
        wid = lax.axis_index("s") * NC + lax.axis_index("c")
        base = wid * CHUNK
        cps = [
            pltpu.async_copy(sd_h.at[:, pl.ds(base, CHUNK)], sd_v, sem),
            pltpu.async_copy(ri_h.at[pl.ds(base, CHUNK)], ri_v, sem),
        ]
        zero = jnp.zeros((LANES,), jnp.float32)

        def zbody(i, c):
            o = i * (LANES * 8)
            for u in range(8):
                acc_v[pl.ds(o + u * LANES, LANES)] = zero
            return c

        lax.fori_loop(0, N_RAYS // (LANES * 8), zbody, 0)
        for cp in cps:
            cp.wait()
        def body(g, c):
            o = g * LANES
            idx = ri_v[pl.ds(o, LANES)]
            v = sd_v[0, pl.ds(o, LANES)]
            plsc.addupdate_scatter(acc_v, [idx], v)
            return c

        lax.fori_loop(0, GROUPS, body, 0, unroll=4)
        pltpu.sync_copy(acc_v, part_h.at[wid])

    @functools.partial(
        pl.kernel,
        out_type=jax.ShapeDtypeStruct((N_RAYS,), jnp.float32),
        mesh=mesh,
        compiler_params=pltpu.CompilerParams(needs_layout_passes=False),
        scratch_types=[
            pltpu.VMEM((NW, RAYS_PER_W), jnp.float32),
            pltpu.VMEM((RAYS_PER_W,), jnp.float32),
        ],
    )
    def sc_finalize(part_h, opac_h, pv, ov):
        wid = lax.axis_index("s") * NC + lax.axis_index("c")
        rbase = wid * RAYS_PER_W
        pltpu.sync_copy(part_h.at[:, pl.ds(rbase, RAYS_PER_W)], pv)

        def body(k, c):
            o = k * LANES
            a = pv[0, pl.ds(o, LANES)]
            for t in range(1, NW):
                a = a + pv[t, pl.ds(o, LANES)]
            ov[pl.ds(o, LANES)] = 1.0 - jnp.exp(-a)
            return c

        lax.fori_loop(0, RAYS_PER_W // LANES, body, 0)
        pltpu.sync_copy(ov, opac_h.at[pl.ds(rbase, RAYS_PER_W)])

    return sc_gather, sc_segsum, sc_finalize


def kernel(rays_o, rays_d, t_starts, t_ends, ray_indices, W1, b1, W2, b2,
           W3, b3):
    sc_gather, sc_segsum, sc_finalize = _sc_kernels()

    d = rays_d / (jnp.linalg.norm(rays_d, axis=-1, keepdims=True) + 1e-8)
    posP = sc_gather(
        rays_o[:, 0], rays_o[:, 1], rays_o[:, 2],
        d[:, 0], d[:, 1], d[:, 2],
        ray_indices, t_starts, t_ends)

    Sts = jnp.asarray(_STS_CONST)
    perm_src = []
    perm_dst = []
    for c in range(3):
        perm_src.append(c)
        perm_dst.append(c)
    for l in range(L_FREQ):
        for c in range(3):
            perm_src.append(3 + 6 * l + c)
            perm_dst.append(8 + 8 * l + c)
            perm_src.append(6 + 6 * l + c)
            perm_dst.append(8 + 8 * l + 4 + c)
    W1t = (jnp.zeros((HIDDEN, D_PE), jnp.float32)
           .at[:, jnp.asarray(perm_dst)].set(W1.T[:, jnp.asarray(perm_src)])
           .at[:, 3].set(b1).astype(jnp.bfloat16))
    W2t = W2.T.astype(jnp.bfloat16)
    W3r = (jnp.zeros((8, HIDDEN), jnp.float32).at[0, :].set(W3[:, 0])
           .astype(jnp.bfloat16))
    sd = _tc_mlp(posP, Sts, W1t, W2t,
                 b2.reshape(HIDDEN, 1).astype(jnp.bfloat16), W3r,
                 b3.reshape(1, 1))

    part = sc_segsum(sd, ray_indices)
    opac = sc_finalize(part)
    return opac.reshape(N_RAYS, 1)


# speedup vs baseline: 1.1051x; 1.1051x over previous
import functools

import numpy as np
import jax
import jax.numpy as jnp
from jax import lax
from jax.experimental import pallas as pl
from jax.experimental.pallas import tpu as pltpu
from jax.experimental.pallas import tpu_sc as plsc

N_RAYS = 4096
N_SAMPLES = 262144
HIDDEN = 128
L_FREQ = 6
D_IN = 3 + 3 * 2 * L_FREQ

LANES = 16
NC = 2
NS = 16
NW = NC * NS
CHUNK = N_SAMPLES // NW
GROUPS = CHUNK // LANES
RAYS_PER_W = N_RAYS // NW

D_PE = 56
BLK = 65536

_SIN_C3 = -0.16665682
_SIN_C5 = 0.0083123785
_SIN_C7 = -1.8492564e-4
_PI_F32 = 3.14159265358979
_INV_PI = 0.3183098861837907


def _fast_sin(a):
    n = jnp.floor(a * _INV_PI + 0.5)
    r = a - n * _PI_F32
    r2 = r * r
    p = _SIN_C7
    p = p * r2 + _SIN_C5
    p = p * r2 + _SIN_C3
    s = (p * r2) * r + r
    sbits = lax.shift_left(n.astype(jnp.int32), 31)
    return lax.bitcast_convert_type(
        lax.bitcast_convert_type(s, jnp.int32) ^ sbits, jnp.float32)


_PH8_CONST = np.array(
    [[0.0], [0.0], [0.0], [0.0],
     [np.pi / 2], [np.pi / 2], [np.pi / 2], [0.0]], np.float32)


def _tc_mlp_body(pos_ref, ph_ref, w1_ref, w2_ref, b2_ref,
                 w3_ref, b3_ref, out_ref):
    P = pos_ref[...]
    row8 = lax.broadcasted_iota(jnp.int32, (8, BLK), 0)
    ph8 = ph_ref[...]
    xyz = jnp.where(row8 < 3, P, 0.0)
    E0 = jnp.where(row8 < 4,
                   jnp.where(row8 < 3, P, 1.0), 0.0)
    A = xyz + jnp.roll(xyz, 4, axis=0) + ph8
    groups = [E0, _fast_sin(A)]
    for _ in range(L_FREQ - 1):
        G = groups[-1]
        Grot = jnp.roll(G, 4, axis=0)
        SS = G * Grot
        C = 1.0 - (G + G) * G
        X = jnp.where(row8 < 4, C, SS + SS)
        groups.append(jnp.roll(X, 4, axis=0))
    E = jnp.concatenate(groups, axis=0)
    h = jnp.maximum(
        jnp.dot(w1_ref[...], E.astype(jnp.bfloat16),
                preferred_element_type=jnp.float32).astype(jnp.bfloat16),
        jnp.bfloat16(0.0))
    h = jnp.maximum(
        jnp.dot(w2_ref[...], h,
                preferred_element_type=jnp.float32).astype(jnp.bfloat16)
        + b2_ref[...], jnp.bfloat16(0.0))
    z8 = jnp.dot(w3_ref[...], h, preferred_element_type=jnp.float32)
    z = z8[0:1, :] + b3_ref[...]
    sp = jnp.maximum(z, 0.0) + jnp.log(1.0 + jnp.exp(-jnp.abs(z)))
    out_ref[...] = sp * P[3:4, :]


def _tc_mlp(posP, ph8, W1t, W2t, b2c, W3r, b3c):
    grid = (N_SAMPLES // BLK,)
    return pl.pallas_call(
        _tc_mlp_body,
        grid=grid,
        in_specs=[
            pl.BlockSpec((8, BLK), lambda i: (0, i)),
            pl.BlockSpec((8, 1), lambda i: (0, 0)),
            pl.BlockSpec((HIDDEN, D_PE), lambda i: (0, 0)),
            pl.BlockSpec((HIDDEN, HIDDEN), lambda i: (0, 0)),
            pl.BlockSpec((HIDDEN, 1), lambda i: (0, 0)),
            pl.BlockSpec((8, HIDDEN), lambda i: (0, 0)),
            pl.BlockSpec((1, 1), lambda i: (0, 0)),
        ],
        out_specs=pl.BlockSpec((1, BLK), lambda i: (0, i)),
        out_shape=jax.ShapeDtypeStruct((1, N_SAMPLES), jnp.float32),
    )(posP, ph8, W1t, W2t, b2c, W3r, b3c)


@functools.cache
def _sc_kernels():
    mesh = plsc.VectorSubcoreMesh(core_axis_name="c", subcore_axis_name="s")

    @functools.partial(
        pl.kernel,
        out_type=jax.ShapeDtypeStruct((8, N_SAMPLES), jnp.float32),
        mesh=mesh,
        compiler_params=pltpu.CompilerParams(needs_layout_passes=False),
        scratch_types=[
            pltpu.VMEM((N_RAYS,), jnp.float32),
            pltpu.VMEM((N_RAYS,), jnp.float32),
            pltpu.VMEM((N_RAYS,), jnp.float32),
            pltpu.VMEM((N_RAYS,), jnp.float32),
            pltpu.VMEM((N_RAYS,), jnp.float32),
            pltpu.VMEM((N_RAYS,), jnp.float32),
            pltpu.VMEM((CHUNK,), jnp.int32),
            pltpu.VMEM((CHUNK,), jnp.float32),
            pltpu.VMEM((CHUNK,), jnp.float32),
            pltpu.VMEM((CHUNK,), jnp.float32),
            pltpu.VMEM((CHUNK,), jnp.float32),
            pltpu.VMEM((CHUNK,), jnp.float32),
            pltpu.VMEM((CHUNK,), jnp.float32),
            pltpu.VMEM((CHUNK,), jnp.float32),
            pltpu.SemaphoreType.DMA,
        ],
    )
    def sc_gather(ox_h, oy_h, oz_h, dx_h, dy_h, dz_h, ri_h, ts_h, te_h,
                  pos_h, ox_v, oy_v, oz_v, dx_v, dy_v, dz_v, ri_v, ts_v,
                  te_v, px_v, py_v, pz_v, dl_v, on_v, sem):
        wid = lax.axis_index("s") * NC + lax.axis_index("c")
        base = wid * CHUNK
        cps = [
            pltpu.async_copy(ox_h, ox_v, sem),
            pltpu.async_copy(oy_h, oy_v, sem),
            pltpu.async_copy(oz_h, oz_v, sem),
            pltpu.async_copy(dx_h, dx_v, sem),
            pltpu.async_copy(dy_h, dy_v, sem),
            pltpu.async_copy(dz_h, dz_v, sem),
            pltpu.async_copy(ri_h.at[pl.ds(base, CHUNK)], ri_v, sem),
            pltpu.async_copy(ts_h.at[pl.ds(base, CHUNK)], ts_v, sem),
            pltpu.async_copy(te_h.at[pl.ds(base, CHUNK)], te_v, sem),
        ]
        one = jnp.full((LANES,), 1.0, jnp.float32)

        def obody(i, c):
            o = i * (LANES * 8)
            for u in range(8):
                on_v[pl.ds(o + u * LANES, LANES)] = one
            return c

        lax.fori_loop(0, CHUNK // (LANES * 8), obody, 0)
        for cp in cps:
            cp.wait()

        def body(g):
            o = g * LANES
            idx = ri_v[pl.ds(o, LANES)]
            tsv = ts_v[pl.ds(o, LANES)]
            tev = te_v[pl.ds(o, LANES)]
            tm = (tsv + tev) * 0.5
            px_v[pl.ds(o, LANES)] = jnp.clip(
                plsc.load_gather(ox_v, [idx])
                + plsc.load_gather(dx_v, [idx]) * tm, -1.0, 1.0)
            py_v[pl.ds(o, LANES)] = jnp.clip(
                plsc.load_gather(oy_v, [idx])
                + plsc.load_gather(dy_v, [idx]) * tm, -1.0, 1.0)
            pz_v[pl.ds(o, LANES)] = jnp.clip(
                plsc.load_gather(oz_v, [idx])
                + plsc.load_gather(dz_v, [idx]) * tm, -1.0, 1.0)
            dl_v[pl.ds(o, LANES)] = tev - tsv

        plsc.parallel_loop(0, GROUPS, 1, unroll=4)(body)
        ops = [
            pltpu.async_copy(px_v, pos_h.at[0, pl.ds(base, CHUNK)], sem),
            pltpu.async_copy(py_v, pos_h.at[1, pl.ds(base, CHUNK)], sem),
            pltpu.async_copy(pz_v, pos_h.at[2, pl.ds(base, CHUNK)], sem),
            pltpu.async_copy(dl_v, pos_h.at[3, pl.ds(base, CHUNK)], sem),
            pltpu.async_copy(on_v, pos_h.at[4, pl.ds(base, CHUNK)], sem),
            pltpu.async_copy(on_v, pos_h.at[5, pl.ds(base, CHUNK)], sem),
            pltpu.async_copy(on_v, pos_h.at[6, pl.ds(base, CHUNK)], sem),
            pltpu.async_copy(on_v, pos_h.at[7, pl.ds(base, CHUNK)], sem),
        ]
        for cp in ops:
            cp.wait()

    @functools.partial(
        pl.kernel,
        out_type=jax.ShapeDtypeStruct((NW, N_RAYS), jnp.float32),
        mesh=mesh,
        compiler_params=pltpu.CompilerParams(needs_layout_passes=False),
        scratch_types=[
            pltpu.VMEM((1, CHUNK), jnp.float32),
            pltpu.VMEM((CHUNK,), jnp.int32),
            pltpu.VMEM((N_RAYS,), jnp.float32),
            pltpu.SemaphoreType.DMA,
        ],
    )
    def sc_segsum(sd_h, ri_h, part_h, sd_v, ri_v, acc_v, sem):
        wid = lax.axis_index("s") * NC + lax.axis_index("c")
        base = wid * CHUNK
        cps = [
            pltpu.async_copy(sd_h.at[:, pl.ds(base, CHUNK)], sd_v, sem),
            pltpu.async_copy(ri_h.at[pl.ds(base, CHUNK)], ri_v, sem),
        ]
        zero = jnp.zeros((LANES,), jnp.float32)

        def zbody(i, c):
            o = i * (LANES * 8)
            for u in range(8):
                acc_v[pl.ds(o + u * LANES, LANES)] = zero
            return c

        lax.fori_loop(0, N_RAYS // (LANES * 8), zbody, 0)
        for cp in cps:
            cp.wait()
        def body(g, c):
            o = g * LANES
            idx = ri_v[pl.ds(o, LANES)]
            v = sd_v[0, pl.ds(o, LANES)]
            plsc.addupdate_scatter(acc_v, [idx], v)
            return c

        lax.fori_loop(0, GROUPS, body, 0, unroll=4)
        pltpu.sync_copy(acc_v, part_h.at[wid])

    @functools.partial(
        pl.kernel,
        out_type=jax.ShapeDtypeStruct((N_RAYS,), jnp.float32),
        mesh=mesh,
        compiler_params=pltpu.CompilerParams(needs_layout_passes=False),
        scratch_types=[
            pltpu.VMEM((NW, RAYS_PER_W), jnp.float32),
            pltpu.VMEM((RAYS_PER_W,), jnp.float32),
        ],
    )
    def sc_finalize(part_h, opac_h, pv, ov):
        wid = lax.axis_index("s") * NC + lax.axis_index("c")
        rbase = wid * RAYS_PER_W
        pltpu.sync_copy(part_h.at[:, pl.ds(rbase, RAYS_PER_W)], pv)

        def body(k, c):
            o = k * LANES
            a = pv[0, pl.ds(o, LANES)]
            for t in range(1, NW):
                a = a + pv[t, pl.ds(o, LANES)]
            ov[pl.ds(o, LANES)] = 1.0 - jnp.exp(-a)
            return c

        lax.fori_loop(0, RAYS_PER_W // LANES, body, 0)
        pltpu.sync_copy(ov, opac_h.at[pl.ds(rbase, RAYS_PER_W)])

    return sc_gather, sc_segsum, sc_finalize


def kernel(rays_o, rays_d, t_starts, t_ends, ray_indices, W1, b1, W2, b2,
           W3, b3):
    sc_gather, sc_segsum, sc_finalize = _sc_kernels()

    d = rays_d / (jnp.linalg.norm(rays_d, axis=-1, keepdims=True) + 1e-8)
    posP = sc_gather(
        rays_o[:, 0], rays_o[:, 1], rays_o[:, 2],
        d[:, 0], d[:, 1], d[:, 2],
        ray_indices, t_starts, t_ends)

    perm_src = []
    perm_dst = []
    for c in range(3):
        perm_src.append(c)
        perm_dst.append(c)
    for l in range(L_FREQ):
        for c in range(3):
            perm_src.append(3 + 6 * l + c)
            perm_dst.append(8 + 8 * l + c)
            perm_src.append(6 + 6 * l + c)
            perm_dst.append(8 + 8 * l + 4 + c)
    W1t = (jnp.zeros((HIDDEN, D_PE), jnp.float32)
           .at[:, jnp.asarray(perm_dst)].set(W1.T[:, jnp.asarray(perm_src)])
           .at[:, 3].set(b1).astype(jnp.bfloat16))
    W2t = W2.T.astype(jnp.bfloat16)
    W3r = (jnp.zeros((8, HIDDEN), jnp.float32).at[0, :].set(W3[:, 0])
           .astype(jnp.bfloat16))
    sd = _tc_mlp(posP, jnp.asarray(_PH8_CONST), W1t, W2t,
                 b2.reshape(HIDDEN, 1).astype(jnp.bfloat16), W3r,
                 b3.reshape(1, 1))

    part = sc_segsum(sd, ray_indices)
    opac = sc_finalize(part)
    return opac.reshape(N_RAYS, 1)


# speedup vs baseline: 1.1181x; 1.0118x over previous
import functools

import numpy as np
import jax
import jax.numpy as jnp
from jax import lax
from jax.experimental import pallas as pl
from jax.experimental.pallas import tpu as pltpu
from jax.experimental.pallas import tpu_sc as plsc

N_RAYS = 4096
N_SAMPLES = 262144
HIDDEN = 128
L_FREQ = 6
D_IN = 3 + 3 * 2 * L_FREQ

LANES = 16
NC = 2
NS = 16
NW = NC * NS
CHUNK = N_SAMPLES // NW
GROUPS = CHUNK // LANES
RAYS_PER_W = N_RAYS // NW

D_PE = 56
BLK = 65536

_SIN_C3 = -0.16665682
_SIN_C5 = 0.0083123785
_SIN_C7 = -1.8492564e-4
_PI_F32 = 3.14159265358979
_INV_PI = 0.3183098861837907


def _fast_sin(a):
    n = jnp.floor(a * _INV_PI + 0.5)
    r = a - n * _PI_F32
    r2 = r * r
    p = _SIN_C7
    p = p * r2 + _SIN_C5
    p = p * r2 + _SIN_C3
    s = (p * r2) * r + r
    sbits = lax.shift_left(n.astype(jnp.int32), 31)
    return lax.bitcast_convert_type(
        lax.bitcast_convert_type(s, jnp.int32) ^ sbits, jnp.float32)


_PH8_CONST = np.array(
    [[0.0], [0.0], [0.0], [0.0],
     [np.pi / 2], [np.pi / 2], [np.pi / 2], [0.0]], np.float32)


def _tc_mlp_body(pos_ref, ph_ref, w1_ref, w2_ref, b2_ref,
                 w3_ref, b3_ref, out_ref):
    P = pos_ref[...]
    row8 = lax.broadcasted_iota(jnp.int32, (8, BLK), 0)
    ph8 = ph_ref[...]
    xyz = jnp.where(row8 < 3, P, 0.0)
    E0 = jnp.where(row8 < 4,
                   jnp.where(row8 < 3, P, 1.0), 0.0)
    A = xyz + jnp.roll(xyz, 4, axis=0) + ph8
    groups = [E0, _fast_sin(A)]
    for _ in range(L_FREQ - 1):
        G = groups[-1]
        Grot = jnp.roll(G, 4, axis=0)
        SS = G * Grot
        C = 1.0 - (G + G) * G
        X = jnp.where(row8 < 4, C, SS + SS)
        groups.append(jnp.roll(X, 4, axis=0))
    E = jnp.concatenate(groups, axis=0)
    h = jnp.maximum(
        jnp.dot(w1_ref[...], E.astype(jnp.bfloat16),
                preferred_element_type=jnp.float32).astype(jnp.bfloat16),
        jnp.bfloat16(0.0))
    h = jnp.maximum(
        jnp.dot(w2_ref[...], h,
                preferred_element_type=jnp.float32).astype(jnp.bfloat16)
        + b2_ref[...], jnp.bfloat16(0.0))
    z8 = jnp.dot(w3_ref[...], h, preferred_element_type=jnp.float32)
    z = z8[0:1, :] + b3_ref[...]
    sp = jnp.maximum(z, 0.0) + jnp.log(1.0 + jnp.exp(-jnp.abs(z)))
    out_ref[...] = sp * P[3:4, :]


def _tc_mlp(posP, ph8, W1t, W2t, b2c, W3r, b3c):
    grid = (N_SAMPLES // BLK,)
    return pl.pallas_call(
        _tc_mlp_body,
        grid=grid,
        in_specs=[
            pl.BlockSpec((8, BLK), lambda i: (0, i)),
            pl.BlockSpec((8, 1), lambda i: (0, 0)),
            pl.BlockSpec((HIDDEN, D_PE), lambda i: (0, 0)),
            pl.BlockSpec((HIDDEN, HIDDEN), lambda i: (0, 0)),
            pl.BlockSpec((HIDDEN, 1), lambda i: (0, 0)),
            pl.BlockSpec((8, HIDDEN), lambda i: (0, 0)),
            pl.BlockSpec((1, 1), lambda i: (0, 0)),
        ],
        out_specs=pl.BlockSpec((1, BLK), lambda i: (0, i)),
        out_shape=jax.ShapeDtypeStruct((1, N_SAMPLES), jnp.float32),
    )(posP, ph8, W1t, W2t, b2c, W3r, b3c)


@functools.cache
def _sc_kernels():
    mesh = plsc.VectorSubcoreMesh(core_axis_name="c", subcore_axis_name="s")

    @functools.partial(
        pl.kernel,
        out_type=jax.ShapeDtypeStruct((8, N_SAMPLES), jnp.float32),
        mesh=mesh,
        compiler_params=pltpu.CompilerParams(needs_layout_passes=False),
        scratch_types=[
            pltpu.VMEM((N_RAYS,), jnp.float32),
            pltpu.VMEM((N_RAYS,), jnp.float32),
            pltpu.VMEM((N_RAYS,), jnp.float32),
            pltpu.VMEM((N_RAYS,), jnp.float32),
            pltpu.VMEM((N_RAYS,), jnp.float32),
            pltpu.VMEM((N_RAYS,), jnp.float32),
            pltpu.VMEM((CHUNK,), jnp.int32),
            pltpu.VMEM((CHUNK,), jnp.float32),
            pltpu.VMEM((CHUNK,), jnp.float32),
            pltpu.VMEM((CHUNK,), jnp.float32),
            pltpu.VMEM((CHUNK,), jnp.float32),
            pltpu.VMEM((CHUNK,), jnp.float32),
            pltpu.VMEM((CHUNK,), jnp.float32),
            pltpu.SemaphoreType.DMA,
        ],
    )
    def sc_gather(ox_h, oy_h, oz_h, dx_h, dy_h, dz_h, ri_h, ts_h, te_h,
                  pos_h, ox_v, oy_v, oz_v, dx_v, dy_v, dz_v, ri_v, ts_v,
                  te_v, px_v, py_v, pz_v, dl_v, sem):
        wid = lax.axis_index("s") * NC + lax.axis_index("c")
        base = wid * CHUNK
        cps = [
            pltpu.async_copy(ox_h, ox_v, sem),
            pltpu.async_copy(oy_h, oy_v, sem),
            pltpu.async_copy(oz_h, oz_v, sem),
            pltpu.async_copy(dx_h, dx_v, sem),
            pltpu.async_copy(dy_h, dy_v, sem),
            pltpu.async_copy(dz_h, dz_v, sem),
            pltpu.async_copy(ri_h.at[pl.ds(base, CHUNK)], ri_v, sem),
            pltpu.async_copy(ts_h.at[pl.ds(base, CHUNK)], ts_v, sem),
            pltpu.async_copy(te_h.at[pl.ds(base, CHUNK)], te_v, sem),
        ]
        for cp in cps:
            cp.wait()

        def body(g):
            o = g * LANES
            idx = ri_v[pl.ds(o, LANES)]
            tsv = ts_v[pl.ds(o, LANES)]
            tev = te_v[pl.ds(o, LANES)]
            tm = (tsv + tev) * 0.5
            px_v[pl.ds(o, LANES)] = jnp.clip(
                plsc.load_gather(ox_v, [idx])
                + plsc.load_gather(dx_v, [idx]) * tm, -1.0, 1.0)
            py_v[pl.ds(o, LANES)] = jnp.clip(
                plsc.load_gather(oy_v, [idx])
                + plsc.load_gather(dy_v, [idx]) * tm, -1.0, 1.0)
            pz_v[pl.ds(o, LANES)] = jnp.clip(
                plsc.load_gather(oz_v, [idx])
                + plsc.load_gather(dz_v, [idx]) * tm, -1.0, 1.0)
            dl_v[pl.ds(o, LANES)] = tev - tsv

        plsc.parallel_loop(0, GROUPS, 1, unroll=4)(body)
        ops = [
            pltpu.async_copy(px_v, pos_h.at[0, pl.ds(base, CHUNK)], sem),
            pltpu.async_copy(py_v, pos_h.at[1, pl.ds(base, CHUNK)], sem),
            pltpu.async_copy(pz_v, pos_h.at[2, pl.ds(base, CHUNK)], sem),
            pltpu.async_copy(dl_v, pos_h.at[3, pl.ds(base, CHUNK)], sem),
        ]
        for cp in ops:
            cp.wait()

    @functools.partial(
        pl.kernel,
        out_type=jax.ShapeDtypeStruct((NW, N_RAYS), jnp.float32),
        mesh=mesh,
        compiler_params=pltpu.CompilerParams(needs_layout_passes=False),
        scratch_types=[
            pltpu.VMEM((1, CHUNK), jnp.float32),
            pltpu.VMEM((CHUNK,), jnp.int32),
            pltpu.VMEM((N_RAYS,), jnp.float32),
            pltpu.SemaphoreType.DMA,
        ],
    )
    def sc_segsum(sd_h, ri_h, part_h, sd_v, ri_v, acc_v, sem):
        wid = lax.axis_index("s") * NC + lax.axis_index("c")
        base = wid * CHUNK
        cps = [
            pltpu.async_copy(sd_h.at[:, pl.ds(base, CHUNK)], sd_v, sem),
            pltpu.async_copy(ri_h.at[pl.ds(base, CHUNK)], ri_v, sem),
        ]
        zero = jnp.zeros((LANES,), jnp.float32)

        def zbody(i, c):
            o = i * (LANES * 8)
            for u in range(8):
                acc_v[pl.ds(o + u * LANES, LANES)] = zero
            return c

        lax.fori_loop(0, N_RAYS // (LANES * 8), zbody, 0)
        for cp in cps:
            cp.wait()
        def body(g, c):
            o = g * LANES
            idx = ri_v[pl.ds(o, LANES)]
            v = sd_v[0, pl.ds(o, LANES)]
            plsc.addupdate_scatter(acc_v, [idx], v)
            return c

        lax.fori_loop(0, GROUPS, body, 0, unroll=4)
        pltpu.sync_copy(acc_v, part_h.at[wid])

    @functools.partial(
        pl.kernel,
        out_type=jax.ShapeDtypeStruct((N_RAYS,), jnp.float32),
        mesh=mesh,
        compiler_params=pltpu.CompilerParams(needs_layout_passes=False),
        scratch_types=[
            pltpu.VMEM((NW, RAYS_PER_W), jnp.float32),
            pltpu.VMEM((RAYS_PER_W,), jnp.float32),
        ],
    )
    def sc_finalize(part_h, opac_h, pv, ov):
        wid = lax.axis_index("s") * NC + lax.axis_index("c")
        rbase = wid * RAYS_PER_W
        pltpu.sync_copy(part_h.at[:, pl.ds(rbase, RAYS_PER_W)], pv)

        def body(k, c):
            o = k * LANES
            a = pv[0, pl.ds(o, LANES)]
            for t in range(1, NW):
                a = a + pv[t, pl.ds(o, LANES)]
            ov[pl.ds(o, LANES)] = 1.0 - jnp.exp(-a)
            return c

        lax.fori_loop(0, RAYS_PER_W // LANES, body, 0)
        pltpu.sync_copy(ov, opac_h.at[pl.ds(rbase, RAYS_PER_W)])

    return sc_gather, sc_segsum, sc_finalize


def kernel(rays_o, rays_d, t_starts, t_ends, ray_indices, W1, b1, W2, b2,
           W3, b3):
    sc_gather, sc_segsum, sc_finalize = _sc_kernels()

    d = rays_d / (jnp.linalg.norm(rays_d, axis=-1, keepdims=True) + 1e-8)
    posP = sc_gather(
        rays_o[:, 0], rays_o[:, 1], rays_o[:, 2],
        d[:, 0], d[:, 1], d[:, 2],
        ray_indices, t_starts, t_ends)

    perm_src = []
    perm_dst = []
    for c in range(3):
        perm_src.append(c)
        perm_dst.append(c)
    for l in range(L_FREQ):
        for c in range(3):
            perm_src.append(3 + 6 * l + c)
            perm_dst.append(8 + 8 * l + c)
            perm_src.append(6 + 6 * l + c)
            perm_dst.append(8 + 8 * l + 4 + c)
    W1t = (jnp.zeros((HIDDEN, D_PE), jnp.float32)
           .at[:, jnp.asarray(perm_dst)].set(W1.T[:, jnp.asarray(perm_src)])
           .at[:, 3].set(b1).astype(jnp.bfloat16))
    W2t = W2.T.astype(jnp.bfloat16)
    W3r = (jnp.zeros((8, HIDDEN), jnp.float32).at[0, :].set(W3[:, 0])
           .astype(jnp.bfloat16))
    sd = _tc_mlp(posP, jnp.asarray(_PH8_CONST), W1t, W2t,
                 b2.reshape(HIDDEN, 1).astype(jnp.bfloat16), W3r,
                 b3.reshape(1, 1))

    part = sc_segsum(sd, ray_indices)
    opac = sc_finalize(part)
    return opac.reshape(N_RAYS, 1)
